# Initial kernel scaffold; baseline (speedup 1.0000x reference)
import jax
import jax.numpy as jnp
from jax.experimental import pallas as pl


def kernel(logits, cur_scores, cur_size, eos_mask, top_beam_outputs, beam_width):
    raise NotImplementedError("write your pallas kernel here")


# speedup vs baseline: 1.8154x; 1.8154x over previous
import functools

import jax
import jax.numpy as jnp
from jax import lax
from jax.experimental import pallas as pl
from jax.experimental.pallas import tpu as pltpu

_LEN_PENALTY_RATIO = 0.8
_ROW_BLOCK = 8
_BIG = 2**30


def _rowtopk_body(k, logits_ref, cur_ref, size_ref, eos_ref, vals_ref, idxs_ref):
    x = logits_ref[...]
    m = jnp.max(x, axis=1, keepdims=True)
    s = jnp.sum(jnp.exp(x - m), axis=1, keepdims=True)
    out = (x - m) - jnp.log(s)
    eos = eos_ref[...]
    out = jnp.where(eos > 0.5, 0.0, out)
    pen = jnp.power((size_ref[...] + 6.0) / 6.0, _LEN_PENALTY_RATIO)
    y = (cur_ref[...] + out) / pen
    iota = lax.broadcasted_iota(jnp.int32, y.shape, 1)
    vals, idxs = [], []
    for _ in range(k):
        mx = jnp.max(y, axis=1, keepdims=True)
        fc = jnp.where(y == mx, iota, _BIG)
        am = jnp.min(fc, axis=1, keepdims=True)
        vals.append(mx)
        idxs.append(am)
        y = jnp.where(iota == am, -jnp.inf, y)
    vals_ref[...] = jnp.concatenate(vals, axis=1)
    idxs_ref[...] = jnp.concatenate(idxs, axis=1)


def _merge_body(k, vocab, seq_len, vals_ref, idxs_ref, tbo_ref, tops_ref, new_ref):
    sc = vals_ref[...]
    wd = idxs_ref[...]
    beam_of_slot = lax.broadcasted_iota(jnp.int32, sc.shape, 1) // k
    fl = beam_of_slot * vocab + wd
    tops, flats = [], []
    for _ in range(k):
        mx = jnp.max(sc, axis=1, keepdims=True)
        fc = jnp.where(sc == mx, fl, _BIG)
        am = jnp.min(fc, axis=1, keepdims=True)
        tops.append(mx)
        flats.append(am)
        sc = jnp.where(fc == am, -jnp.inf, sc)
    tops_ref[...] = jnp.concatenate(tops, axis=1)
    flat = jnp.concatenate(flats, axis=1)
    beam = flat // vocab
    word = flat - beam * vocab
    tbo = tbo_ref[...]
    for kk in range(k):
        bsel = beam[:, kk : kk + 1]
        chunk = jnp.zeros((tbo.shape[0], seq_len), jnp.int32)
        for j in range(k):
            row = tbo[:, j * seq_len : (j + 1) * seq_len]
            chunk = jnp.where(bsel == j, row, chunk)
        new_ref[:, pl.ds(kk * (seq_len + 1), seq_len)] = chunk
        new_ref[:, pl.ds(kk * (seq_len + 1) + seq_len, 1)] = word[:, kk : kk + 1]


def kernel(logits, cur_scores, cur_size, eos_mask, top_beam_outputs, beam_width):
    Bb, k, seq_len = top_beam_outputs.shape
    rows, vocab = logits.shape
    rb = _ROW_BLOCK
    grid = rows // rb

    size_f = cur_size.astype(jnp.float32).reshape(rows, 1)
    eos_f = eos_mask.astype(jnp.float32).reshape(rows, 1)

    vals, idxs = pl.pallas_call(
        functools.partial(_rowtopk_body, k),
        grid=(grid,),
        in_specs=[
            pl.BlockSpec((rb, vocab), lambda i: (i, 0)),
            pl.BlockSpec((rb, 1), lambda i: (i, 0)),
            pl.BlockSpec((rb, 1), lambda i: (i, 0)),
            pl.BlockSpec((rb, 1), lambda i: (i, 0)),
        ],
        out_specs=[
            pl.BlockSpec((rb, k), lambda i: (i, 0)),
            pl.BlockSpec((rb, k), lambda i: (i, 0)),
        ],
        out_shape=[
            jax.ShapeDtypeStruct((rows, k), jnp.float32),
            jax.ShapeDtypeStruct((rows, k), jnp.int32),
        ],
    )(logits, cur_scores, size_f, eos_f)

    vals2 = vals.reshape(Bb, k * k)
    idxs2 = idxs.reshape(Bb, k * k)
    tbo2 = top_beam_outputs.reshape(Bb, k * seq_len).astype(jnp.int32)

    tops, new2 = pl.pallas_call(
        functools.partial(_merge_body, k, vocab, seq_len),
        out_shape=[
            jax.ShapeDtypeStruct((Bb, k), jnp.float32),
            jax.ShapeDtypeStruct((Bb, k * (seq_len + 1)), jnp.int32),
        ],
    )(vals2, idxs2, tbo2)

    top_scores = tops + 0.0 * beam_width
    new_outputs = new2.reshape(Bb, k, seq_len + 1).astype(top_beam_outputs.dtype)
    return top_scores, new_outputs


# speedup vs baseline: 2.2038x; 1.2139x over previous
import functools

import jax
import jax.numpy as jnp
from jax import lax
from jax.experimental import pallas as pl
from jax.experimental.pallas import tpu as pltpu

_LEN_PENALTY_RATIO = 0.8
_ROW_BLOCK = 8
_BIG = 2**30


def _rowtopk_body(k, logits_ref, vals_ref, idxs_ref, sum_ref):
    x = logits_ref[...]
    iota = lax.broadcasted_iota(jnp.int32, x.shape, 1)
    v = jnp.max(x, axis=1, keepdims=True)
    sum_ref[...] = jnp.sum(jnp.exp(x - v), axis=1, keepdims=True)
    a = jnp.min(jnp.where(x == v, iota, _BIG), axis=1, keepdims=True)
    vals, idxs = [v], [a]
    y = x
    for _ in range(k - 1):
        y = jnp.where(iota == a, -jnp.inf, y)
        v = jnp.max(y, axis=1, keepdims=True)
        a = jnp.min(jnp.where(y == v, iota, _BIG), axis=1, keepdims=True)
        vals.append(v)
        idxs.append(a)
    vals_ref[...] = jnp.concatenate(vals, axis=1)
    idxs_ref[...] = jnp.concatenate(idxs, axis=1)


def _merge_body(k, vocab, seq_len, vals_ref, idxs_ref, sums_ref, cur_ref,
                size_ref, eos_ref, tbo_ref, tops_ref, new_ref):
    val = vals_ref[...]
    wd = idxs_ref[...]
    Bb = val.shape[0]

    def rep(col):
        return jnp.concatenate(
            [jnp.broadcast_to(col[:, g : g + 1], (Bb, k)) for g in range(k)], axis=1)

    m_r = jnp.concatenate(
        [jnp.broadcast_to(val[:, g * k : g * k + 1], (Bb, k)) for g in range(k)],
        axis=1)
    s_r = rep(sums_ref[...])
    cur_r = rep(cur_ref[...])
    size_r = rep(size_ref[...])
    eos_r = rep(eos_ref[...])

    out = (val - m_r) - jnp.log(s_r)
    out = jnp.where(eos_r > 0.5, 0.0, out)
    pen = jnp.power((size_r + 6.0) / 6.0, _LEN_PENALTY_RATIO)
    sc = (cur_r + out) / pen

    slot = lax.broadcasted_iota(jnp.int32, sc.shape, 1)
    beam_of_slot = slot // k
    wd = jnp.where(eos_r > 0.5, slot - beam_of_slot * k, wd)
    fl = beam_of_slot * vocab + wd

    tops, flats = [], []
    for _ in range(k):
        mx = jnp.max(sc, axis=1, keepdims=True)
        fc = jnp.where(sc == mx, fl, _BIG)
        am = jnp.min(fc, axis=1, keepdims=True)
        tops.append(mx)
        flats.append(am)
        sc = jnp.where(fc == am, -jnp.inf, sc)
    tops_ref[...] = jnp.concatenate(tops, axis=1)
    flat = jnp.concatenate(flats, axis=1)
    beam = flat // vocab
    word = flat - beam * vocab
    tbo = tbo_ref[...]
    for kk in range(k):
        bsel = beam[:, kk : kk + 1]
        chunk = jnp.zeros((Bb, seq_len), jnp.int32)
        for j in range(k):
            row = tbo[:, j * seq_len : (j + 1) * seq_len]
            chunk = jnp.where(bsel == j, row, chunk)
        new_ref[:, pl.ds(kk * (seq_len + 1), seq_len)] = chunk
        new_ref[:, pl.ds(kk * (seq_len + 1) + seq_len, 1)] = word[:, kk : kk + 1]


def kernel(logits, cur_scores, cur_size, eos_mask, top_beam_outputs, beam_width):
    Bb, k, seq_len = top_beam_outputs.shape
    rows, vocab = logits.shape
    rb = _ROW_BLOCK
    grid = rows // rb

    vals, idxs, sums = pl.pallas_call(
        functools.partial(_rowtopk_body, k),
        grid=(grid,),
        in_specs=[pl.BlockSpec((rb, vocab), lambda i: (i, 0))],
        out_specs=[
            pl.BlockSpec((rb, k), lambda i: (i, 0)),
            pl.BlockSpec((rb, k), lambda i: (i, 0)),
            pl.BlockSpec((rb, 1), lambda i: (i, 0)),
        ],
        out_shape=[
            jax.ShapeDtypeStruct((rows, k), jnp.float32),
            jax.ShapeDtypeStruct((rows, k), jnp.int32),
            jax.ShapeDtypeStruct((rows, 1), jnp.float32),
        ],
    )(logits)

    vals2 = vals.reshape(Bb, k * k)
    idxs2 = idxs.reshape(Bb, k * k)
    sums2 = sums.reshape(Bb, k)
    cur2 = cur_scores.reshape(Bb, k)
    size2 = cur_size.astype(jnp.float32).reshape(Bb, k)
    eos2 = eos_mask.astype(jnp.float32).reshape(Bb, k)
    tbo2 = top_beam_outputs.reshape(Bb, k * seq_len).astype(jnp.int32)

    tops, new2 = pl.pallas_call(
        functools.partial(_merge_body, k, vocab, seq_len),
        out_shape=[
            jax.ShapeDtypeStruct((Bb, k), jnp.float32),
            jax.ShapeDtypeStruct((Bb, k * (seq_len + 1)), jnp.int32),
        ],
    )(vals2, idxs2, sums2, cur2, size2, eos2, tbo2)

    top_scores = tops + 0.0 * beam_width
    new_outputs = new2.reshape(Bb, k, seq_len + 1).astype(top_beam_outputs.dtype)
    return top_scores, new_outputs


# speedup vs baseline: 3.0226x; 1.3715x over previous
import functools

import jax
import jax.numpy as jnp
from jax import lax
from jax.experimental import pallas as pl
from jax.experimental.pallas import tpu as pltpu

_LEN_PENALTY_RATIO = 0.8
_ROW_BLOCK = 8
_BIG = 2**30


def _rowtopk_body(k, logits_ref, vals_ref, idxs_ref, sum_ref):
    x = logits_ref[...]
    iota = lax.broadcasted_iota(jnp.int32, x.shape, 1)
    v = jnp.max(x, axis=1, keepdims=True)
    sum_ref[...] = jnp.sum(jnp.exp(x - v), axis=1, keepdims=True)
    a = jnp.argmax(x, axis=1, keepdims=True).astype(jnp.int32)
    vals, idxs = [v], [a]
    y = x
    for _ in range(k - 1):
        y = jnp.where(iota == a, -jnp.inf, y)
        v = jnp.max(y, axis=1, keepdims=True)
        a = jnp.argmax(y, axis=1, keepdims=True).astype(jnp.int32)
        vals.append(v)
        idxs.append(a)
    vals_ref[...] = jnp.concatenate(vals, axis=1)
    idxs_ref[...] = jnp.concatenate(idxs, axis=1)


def _merge_body(k, vocab, seq_len, vals_ref, idxs_ref, sums_ref, cur_ref,
                size_ref, eos_ref, tbo_ref, tops_ref, new_ref):
    val = vals_ref[...]
    wd = idxs_ref[...]
    Bb = val.shape[0]

    def rep(col):
        return jnp.concatenate(
            [jnp.broadcast_to(col[:, g : g + 1], (Bb, k)) for g in range(k)], axis=1)

    m_r = jnp.concatenate(
        [jnp.broadcast_to(val[:, g * k : g * k + 1], (Bb, k)) for g in range(k)],
        axis=1)
    s_r = rep(sums_ref[...])
    cur_r = rep(cur_ref[...])
    size_r = rep(size_ref[...])
    eos_r = rep(eos_ref[...])

    out = (val - m_r) - jnp.log(s_r)
    out = jnp.where(eos_r > 0.5, 0.0, out)
    pen = jnp.power((size_r + 6.0) / 6.0, _LEN_PENALTY_RATIO)
    sc = (cur_r + out) / pen

    slot = lax.broadcasted_iota(jnp.int32, sc.shape, 1)
    beam_of_slot = slot // k
    wd = jnp.where(eos_r > 0.5, slot - beam_of_slot * k, wd)
    fl = beam_of_slot * vocab + wd

    tops, flats = [], []
    for _ in range(k):
        mx = jnp.max(sc, axis=1, keepdims=True)
        fc = jnp.where(sc == mx, fl, _BIG)
        am = jnp.min(fc, axis=1, keepdims=True)
        tops.append(mx)
        flats.append(am)
        sc = jnp.where(fc == am, -jnp.inf, sc)
    tops_ref[...] = jnp.concatenate(tops, axis=1)
    flat = jnp.concatenate(flats, axis=1)
    beam = flat // vocab
    word = flat - beam * vocab
    tbo = tbo_ref[...]
    for kk in range(k):
        bsel = beam[:, kk : kk + 1]
        chunk = jnp.zeros((Bb, seq_len), jnp.int32)
        for j in range(k):
            row = tbo[:, j * seq_len : (j + 1) * seq_len]
            chunk = jnp.where(bsel == j, row, chunk)
        new_ref[:, pl.ds(kk * (seq_len + 1), seq_len)] = chunk
        new_ref[:, pl.ds(kk * (seq_len + 1) + seq_len, 1)] = word[:, kk : kk + 1]


def kernel(logits, cur_scores, cur_size, eos_mask, top_beam_outputs, beam_width):
    Bb, k, seq_len = top_beam_outputs.shape
    rows, vocab = logits.shape
    rb = _ROW_BLOCK
    grid = rows // rb

    vals, idxs, sums = pl.pallas_call(
        functools.partial(_rowtopk_body, k),
        grid=(grid,),
        in_specs=[pl.BlockSpec((rb, vocab), lambda i: (i, 0))],
        out_specs=[
            pl.BlockSpec((rb, k), lambda i: (i, 0)),
            pl.BlockSpec((rb, k), lambda i: (i, 0)),
            pl.BlockSpec((rb, 1), lambda i: (i, 0)),
        ],
        out_shape=[
            jax.ShapeDtypeStruct((rows, k), jnp.float32),
            jax.ShapeDtypeStruct((rows, k), jnp.int32),
            jax.ShapeDtypeStruct((rows, 1), jnp.float32),
        ],
    )(logits)

    vals2 = vals.reshape(Bb, k * k)
    idxs2 = idxs.reshape(Bb, k * k)
    sums2 = sums.reshape(Bb, k)
    cur2 = cur_scores.reshape(Bb, k)
    size2 = cur_size.astype(jnp.float32).reshape(Bb, k)
    eos2 = eos_mask.astype(jnp.float32).reshape(Bb, k)
    tbo2 = top_beam_outputs.reshape(Bb, k * seq_len).astype(jnp.int32)

    tops, new2 = pl.pallas_call(
        functools.partial(_merge_body, k, vocab, seq_len),
        out_shape=[
            jax.ShapeDtypeStruct((Bb, k), jnp.float32),
            jax.ShapeDtypeStruct((Bb, k * (seq_len + 1)), jnp.int32),
        ],
    )(vals2, idxs2, sums2, cur2, size2, eos2, tbo2)

    top_scores = tops + 0.0 * beam_width
    new_outputs = new2.reshape(Bb, k, seq_len + 1).astype(top_beam_outputs.dtype)
    return top_scores, new_outputs


# speedup vs baseline: 3.0881x; 1.0217x over previous
import functools

import jax
import jax.numpy as jnp
from jax import lax
from jax.experimental import pallas as pl
from jax.experimental.pallas import tpu as pltpu

_LEN_PENALTY_RATIO = 0.8
_ROW_BLOCK = 8
_BIG = 2**30


def _rowtopk_body(k, rb, logits_ref, vals_ref, idxs_ref, sum_ref):
    x = logits_ref[...]
    v = jnp.max(x, axis=1, keepdims=True)
    sum_ref[...] = jnp.sum(jnp.exp(x - v), axis=1, keepdims=True)
    a = jnp.argmax(x, axis=1, keepdims=True).astype(jnp.int32)
    lane = lax.broadcasted_iota(jnp.int32, (1, 128), 1)
    vals, idxs = [v], [a]
    for i in range(1, k):
        for r in range(rb):
            ar = a[r, 0]
            base = pl.multiple_of((ar // 128) * 128, 128)
            off = ar - base
            chunk = logits_ref[pl.ds(r, 1), pl.ds(base, 128)]
            logits_ref[pl.ds(r, 1), pl.ds(base, 128)] = jnp.where(
                lane == off, -jnp.inf, chunk)
        a = jnp.argmax(logits_ref[...], axis=1, keepdims=True).astype(jnp.int32)
        idxs.append(a)
        vcol = []
        for r in range(rb):
            ar = a[r, 0]
            base = pl.multiple_of((ar // 128) * 128, 128)
            off = ar - base
            chunk = logits_ref[pl.ds(r, 1), pl.ds(base, 128)]
            vcol.append(jnp.max(jnp.where(lane == off, chunk, -jnp.inf),
                                axis=1, keepdims=True))
        vals.append(jnp.concatenate(vcol, axis=0))
    vals_ref[...] = jnp.concatenate(vals, axis=1)
    idxs_ref[...] = jnp.concatenate(idxs, axis=1)


def _merge_body(k, vocab, seq_len, vals_ref, idxs_ref, sums_ref, cur_ref,
                size_ref, eos_ref, tbo_ref, tops_ref, new_ref):
    val = vals_ref[...]
    wd = idxs_ref[...]
    Bb = val.shape[0]

    def rep(col):
        return jnp.concatenate(
            [jnp.broadcast_to(col[:, g : g + 1], (Bb, k)) for g in range(k)], axis=1)

    m_r = jnp.concatenate(
        [jnp.broadcast_to(val[:, g * k : g * k + 1], (Bb, k)) for g in range(k)],
        axis=1)
    s_r = rep(sums_ref[...])
    cur_r = rep(cur_ref[...])
    size_r = rep(size_ref[...])
    eos_r = rep(eos_ref[...])

    out = (val - m_r) - jnp.log(s_r)
    out = jnp.where(eos_r > 0.5, 0.0, out)
    pen = jnp.power((size_r + 6.0) / 6.0, _LEN_PENALTY_RATIO)
    sc = (cur_r + out) / pen

    slot = lax.broadcasted_iota(jnp.int32, sc.shape, 1)
    beam_of_slot = slot // k
    wd = jnp.where(eos_r > 0.5, slot - beam_of_slot * k, wd)
    fl = beam_of_slot * vocab + wd

    tops, flats = [], []
    for _ in range(k):
        mx = jnp.max(sc, axis=1, keepdims=True)
        fc = jnp.where(sc == mx, fl, _BIG)
        am = jnp.min(fc, axis=1, keepdims=True)
        tops.append(mx)
        flats.append(am)
        sc = jnp.where(fc == am, -jnp.inf, sc)
    tops_ref[...] = jnp.concatenate(tops, axis=1)
    flat = jnp.concatenate(flats, axis=1)
    beam = flat // vocab
    word = flat - beam * vocab
    tbo = tbo_ref[...]
    for kk in range(k):
        bsel = beam[:, kk : kk + 1]
        chunk = jnp.zeros((Bb, seq_len), jnp.int32)
        for j in range(k):
            row = tbo[:, j * seq_len : (j + 1) * seq_len]
            chunk = jnp.where(bsel == j, row, chunk)
        new_ref[:, pl.ds(kk * (seq_len + 1), seq_len)] = chunk
        new_ref[:, pl.ds(kk * (seq_len + 1) + seq_len, 1)] = word[:, kk : kk + 1]


def kernel(logits, cur_scores, cur_size, eos_mask, top_beam_outputs, beam_width):
    Bb, k, seq_len = top_beam_outputs.shape
    rows, vocab = logits.shape
    rb = _ROW_BLOCK
    grid = rows // rb

    vals, idxs, sums = pl.pallas_call(
        functools.partial(_rowtopk_body, k, rb),
        grid=(grid,),
        in_specs=[pl.BlockSpec((rb, vocab), lambda i: (i, 0))],
        out_specs=[
            pl.BlockSpec((rb, k), lambda i: (i, 0)),
            pl.BlockSpec((rb, k), lambda i: (i, 0)),
            pl.BlockSpec((rb, 1), lambda i: (i, 0)),
        ],
        out_shape=[
            jax.ShapeDtypeStruct((rows, k), jnp.float32),
            jax.ShapeDtypeStruct((rows, k), jnp.int32),
            jax.ShapeDtypeStruct((rows, 1), jnp.float32),
        ],
    )(logits)

    vals2 = vals.reshape(Bb, k * k)
    idxs2 = idxs.reshape(Bb, k * k)
    sums2 = sums.reshape(Bb, k)
    cur2 = cur_scores.reshape(Bb, k)
    size2 = cur_size.astype(jnp.float32).reshape(Bb, k)
    eos2 = eos_mask.astype(jnp.float32).reshape(Bb, k)
    tbo2 = top_beam_outputs.reshape(Bb, k * seq_len).astype(jnp.int32)

    tops, new2 = pl.pallas_call(
        functools.partial(_merge_body, k, vocab, seq_len),
        out_shape=[
            jax.ShapeDtypeStruct((Bb, k), jnp.float32),
            jax.ShapeDtypeStruct((Bb, k * (seq_len + 1)), jnp.int32),
        ],
    )(vals2, idxs2, sums2, cur2, size2, eos2, tbo2)

    top_scores = tops + 0.0 * beam_width
    new_outputs = new2.reshape(Bb, k, seq_len + 1).astype(top_beam_outputs.dtype)
    return top_scores, new_outputs
